# baseline (speedup 1.0000x reference)
import jax
import jax.numpy as jnp
from jax.experimental import pallas as pl
from jax.experimental.pallas import tpu as pltpu

EPS = 1e-12
N_SKILLS = 8
RANK = 16
NCHUNK = 8


def _fused_body(tid_ref, ml_ref, x_ref, w_ref, bias_ref, la_ref, lb_ref,
                out_ref, xbuf, obuf, weff_ref, lsem, ssem):
    bsz, seq, _ = x_ref.shape
    ck = seq // NCHUNK

    def load_chunk(b, c):
        pltpu.make_async_copy(
            x_ref.at[b, pl.ds(c * ck, ck)],
            xbuf.at[b % 2, pl.ds(c * ck, ck)], lsem.at[b % 2, c]).start()

    def load_wait_chunk(b, c):
        pltpu.make_async_copy(
            x_ref.at[b, pl.ds(c * ck, ck)],
            xbuf.at[b % 2, pl.ds(c * ck, ck)], lsem.at[b % 2, c]).wait()

    def store_chunk(b, c):
        pltpu.make_async_copy(
            obuf.at[b % 2, pl.ds(c * ck, ck)],
            out_ref.at[b, pl.ds(c * ck, ck)], ssem.at[b % 2, c]).start()

    def store_wait_chunk(b, c):
        pltpu.make_async_copy(
            obuf.at[b % 2, pl.ds(c * ck, ck)],
            out_ref.at[b, pl.ds(c * ck, ck)], ssem.at[b % 2, c]).wait()

    for c in range(NCHUNK):
        load_chunk(0, c)
    for b in range(bsz):
        if b + 1 < bsz:
            for c in range(NCHUNK):
                load_chunk(b + 1, c)
        tid = tid_ref[b]
        sig = [1.0 / (1.0 + jnp.exp(-ml_ref[tid, k])) for k in range(N_SKILLS)]
        tot = sig[0]
        for k in range(1, N_SKILLS):
            tot = tot + sig[k]
        inv = 1.0 / (tot + EPS)
        wt = [s * inv for s in sig]
        A = la_ref[0] * wt[0]
        Bm = lb_ref[0] * wt[0]
        for k in range(1, N_SKILLS):
            A = A + la_ref[k] * wt[k]
            Bm = Bm + lb_ref[k] * wt[k]
        abT = jax.lax.dot_general(
            Bm, A, (((0,), (1,)), ((), ())),
            preferred_element_type=jnp.float32)
        weff_ref[...] = w_ref[...] + abT * (1.0 / RANK)

        for c in range(NCHUNK):
            load_wait_chunk(b, c)
            if b >= 2:
                store_wait_chunk(b - 2, c)
            obuf[b % 2, pl.ds(c * ck, ck)] = jax.lax.dot_general(
                xbuf[b % 2, pl.ds(c * ck, ck)], weff_ref[...],
                (((1,), (1,)), ((), ())),
                preferred_element_type=jnp.float32) + bias_ref[...]
            store_chunk(b, c)
    for b in range(max(0, bsz - 2), bsz):
        for c in range(NCHUNK):
            store_wait_chunk(b, c)


def kernel(x, task_ids, module_logits, weight, bias, lora_a, lora_b):
    bsz, seq, in_f = x.shape
    out_f = weight.shape[0]
    n_tasks, n_sk = module_logits.shape

    bias2 = bias.reshape(1, out_f)
    la = lora_a.reshape(n_sk, in_f, RANK)
    lb = lora_b.reshape(n_sk, RANK, out_f)

    out = pl.pallas_call(
        _fused_body,
        in_specs=[
            pl.BlockSpec(memory_space=pltpu.SMEM),
            pl.BlockSpec(memory_space=pltpu.SMEM),
            pl.BlockSpec(memory_space=pl.ANY),
            pl.BlockSpec(memory_space=pltpu.VMEM),
            pl.BlockSpec(memory_space=pltpu.VMEM),
            pl.BlockSpec(memory_space=pltpu.VMEM),
            pl.BlockSpec(memory_space=pltpu.VMEM),
        ],
        out_specs=pl.BlockSpec(memory_space=pl.ANY),
        out_shape=jax.ShapeDtypeStruct((bsz, seq, out_f), jnp.float32),
        scratch_shapes=[
            pltpu.VMEM((2, seq, in_f), jnp.float32),
            pltpu.VMEM((2, seq, out_f), jnp.float32),
            pltpu.VMEM((out_f, in_f), jnp.float32),
            pltpu.SemaphoreType.DMA((2, NCHUNK)),
            pltpu.SemaphoreType.DMA((2, NCHUNK)),
        ],
    )(task_ids.astype(jnp.int32), module_logits, x, weight, bias2, la, lb)
    return out


# speedup vs baseline: 1.1655x; 1.1655x over previous
import jax
import jax.numpy as jnp
from jax.experimental import pallas as pl
from jax.experimental.pallas import tpu as pltpu

EPS = 1e-12
N_SKILLS = 8
RANK = 16
NCHUNK = 2


def _fused_body(tid_ref, ml_ref, x_ref, w_ref, bias_ref, la_ref, lb_ref,
                out_ref, xbuf, obuf, weff_ref, lsem, ssem):
    bsz, seq, _ = x_ref.shape
    ck = seq // NCHUNK

    def load_chunk(b, c):
        pltpu.make_async_copy(
            x_ref.at[b, pl.ds(c * ck, ck)],
            xbuf.at[b % 2, pl.ds(c * ck, ck)], lsem.at[b % 2, c]).start()

    def load_wait_chunk(b, c):
        pltpu.make_async_copy(
            x_ref.at[b, pl.ds(c * ck, ck)],
            xbuf.at[b % 2, pl.ds(c * ck, ck)], lsem.at[b % 2, c]).wait()

    def store_chunk(b, c):
        pltpu.make_async_copy(
            obuf.at[b % 2, pl.ds(c * ck, ck)],
            out_ref.at[b, pl.ds(c * ck, ck)], ssem.at[b % 2, c]).start()

    def store_wait_chunk(b, c):
        pltpu.make_async_copy(
            obuf.at[b % 2, pl.ds(c * ck, ck)],
            out_ref.at[b, pl.ds(c * ck, ck)], ssem.at[b % 2, c]).wait()

    for c in range(NCHUNK):
        load_chunk(0, c)
    for b in range(bsz):
        if b + 1 < bsz:
            for c in range(NCHUNK):
                load_chunk(b + 1, c)
        tid = tid_ref[b]
        sig = [1.0 / (1.0 + jnp.exp(-ml_ref[tid, k])) for k in range(N_SKILLS)]
        tot = sig[0]
        for k in range(1, N_SKILLS):
            tot = tot + sig[k]
        inv = 1.0 / (tot + EPS)
        wt = [s * inv for s in sig]
        A = la_ref[0] * wt[0]
        Bm = lb_ref[0] * wt[0]
        for k in range(1, N_SKILLS):
            A = A + la_ref[k] * wt[k]
            Bm = Bm + lb_ref[k] * wt[k]
        abT = jax.lax.dot_general(
            Bm, A, (((0,), (1,)), ((), ())),
            preferred_element_type=jnp.float32)
        weff_ref[...] = w_ref[...] + abT * (1.0 / RANK)

        for c in range(NCHUNK):
            load_wait_chunk(b, c)
            if b >= 2:
                store_wait_chunk(b - 2, c)
            obuf[b % 2, pl.ds(c * ck, ck)] = jax.lax.dot_general(
                xbuf[b % 2, pl.ds(c * ck, ck)], weff_ref[...],
                (((1,), (1,)), ((), ())),
                preferred_element_type=jnp.float32) + bias_ref[...]
            store_chunk(b, c)
    for b in range(max(0, bsz - 2), bsz):
        for c in range(NCHUNK):
            store_wait_chunk(b, c)


def kernel(x, task_ids, module_logits, weight, bias, lora_a, lora_b):
    bsz, seq, in_f = x.shape
    out_f = weight.shape[0]
    n_tasks, n_sk = module_logits.shape

    bias2 = bias.reshape(1, out_f)
    la = lora_a.reshape(n_sk, in_f, RANK)
    lb = lora_b.reshape(n_sk, RANK, out_f)

    out = pl.pallas_call(
        _fused_body,
        in_specs=[
            pl.BlockSpec(memory_space=pltpu.SMEM),
            pl.BlockSpec(memory_space=pltpu.SMEM),
            pl.BlockSpec(memory_space=pl.ANY),
            pl.BlockSpec(memory_space=pltpu.VMEM),
            pl.BlockSpec(memory_space=pltpu.VMEM),
            pl.BlockSpec(memory_space=pltpu.VMEM),
            pl.BlockSpec(memory_space=pltpu.VMEM),
        ],
        out_specs=pl.BlockSpec(memory_space=pl.ANY),
        out_shape=jax.ShapeDtypeStruct((bsz, seq, out_f), jnp.float32),
        scratch_shapes=[
            pltpu.VMEM((2, seq, in_f), jnp.float32),
            pltpu.VMEM((2, seq, out_f), jnp.float32),
            pltpu.VMEM((out_f, in_f), jnp.float32),
            pltpu.SemaphoreType.DMA((2, NCHUNK)),
            pltpu.SemaphoreType.DMA((2, NCHUNK)),
        ],
    )(task_ids.astype(jnp.int32), module_logits, x, weight, bias2, la, lb)
    return out


# speedup vs baseline: 1.1668x; 1.0011x over previous
import jax
import jax.numpy as jnp
from jax.experimental import pallas as pl
from jax.experimental.pallas import tpu as pltpu

EPS = 1e-12
N_SKILLS = 8
RANK = 16
NCHUNK = 2


def _fused_body(tid_ref, ml_ref, x_ref, w_ref, bias_ref, la_ref, lb_ref,
                out_ref, xbuf, obuf, weff_ref, lsem, ssem):
    bsz, seq, _ = x_ref.shape
    ck = seq // NCHUNK

    hk = ck // 2

    def load_chunk(b, c):
        for u in range(2):
            pltpu.make_async_copy(
                x_ref.at[b, pl.ds(c * ck + u * hk, hk)],
                xbuf.at[b % 2, pl.ds(c * ck + u * hk, hk)],
                lsem.at[b % 2, c, u]).start()

    def load_wait_chunk(b, c):
        for u in range(2):
            pltpu.make_async_copy(
                x_ref.at[b, pl.ds(c * ck + u * hk, hk)],
                xbuf.at[b % 2, pl.ds(c * ck + u * hk, hk)],
                lsem.at[b % 2, c, u]).wait()

    def store_chunk(b, c):
        for u in range(2):
            pltpu.make_async_copy(
                obuf.at[b % 2, pl.ds(c * ck + u * hk, hk)],
                out_ref.at[b, pl.ds(c * ck + u * hk, hk)],
                ssem.at[b % 2, c, u]).start()

    def store_wait_chunk(b, c):
        for u in range(2):
            pltpu.make_async_copy(
                obuf.at[b % 2, pl.ds(c * ck + u * hk, hk)],
                out_ref.at[b, pl.ds(c * ck + u * hk, hk)],
                ssem.at[b % 2, c, u]).wait()

    for c in range(NCHUNK):
        load_chunk(0, c)
    for b in range(bsz):
        if b + 1 < bsz:
            for c in range(NCHUNK):
                load_chunk(b + 1, c)
        tid = tid_ref[b]
        sig = [1.0 / (1.0 + jnp.exp(-ml_ref[tid, k])) for k in range(N_SKILLS)]
        tot = sig[0]
        for k in range(1, N_SKILLS):
            tot = tot + sig[k]
        inv = 1.0 / (tot + EPS)
        wt = [s * inv for s in sig]
        A = la_ref[0] * wt[0]
        Bm = lb_ref[0] * wt[0]
        for k in range(1, N_SKILLS):
            A = A + la_ref[k] * wt[k]
            Bm = Bm + lb_ref[k] * wt[k]
        abT = jax.lax.dot_general(
            Bm, A, (((0,), (1,)), ((), ())),
            preferred_element_type=jnp.float32)
        weff_ref[...] = w_ref[...] + abT * (1.0 / RANK)

        for c in range(NCHUNK):
            load_wait_chunk(b, c)
            if b >= 2:
                store_wait_chunk(b - 2, c)
            obuf[b % 2, pl.ds(c * ck, ck)] = jax.lax.dot_general(
                xbuf[b % 2, pl.ds(c * ck, ck)], weff_ref[...],
                (((1,), (1,)), ((), ())),
                preferred_element_type=jnp.float32) + bias_ref[...]
            store_chunk(b, c)
    for b in range(max(0, bsz - 2), bsz):
        for c in range(NCHUNK):
            store_wait_chunk(b, c)


def kernel(x, task_ids, module_logits, weight, bias, lora_a, lora_b):
    bsz, seq, in_f = x.shape
    out_f = weight.shape[0]
    n_tasks, n_sk = module_logits.shape

    bias2 = bias.reshape(1, out_f)
    la = lora_a.reshape(n_sk, in_f, RANK)
    lb = lora_b.reshape(n_sk, RANK, out_f)

    out = pl.pallas_call(
        _fused_body,
        in_specs=[
            pl.BlockSpec(memory_space=pltpu.SMEM),
            pl.BlockSpec(memory_space=pltpu.SMEM),
            pl.BlockSpec(memory_space=pl.ANY),
            pl.BlockSpec(memory_space=pltpu.VMEM),
            pl.BlockSpec(memory_space=pltpu.VMEM),
            pl.BlockSpec(memory_space=pltpu.VMEM),
            pl.BlockSpec(memory_space=pltpu.VMEM),
        ],
        out_specs=pl.BlockSpec(memory_space=pl.ANY),
        out_shape=jax.ShapeDtypeStruct((bsz, seq, out_f), jnp.float32),
        scratch_shapes=[
            pltpu.VMEM((2, seq, in_f), jnp.float32),
            pltpu.VMEM((2, seq, out_f), jnp.float32),
            pltpu.VMEM((out_f, in_f), jnp.float32),
            pltpu.SemaphoreType.DMA((2, NCHUNK, 2)),
            pltpu.SemaphoreType.DMA((2, NCHUNK, 2)),
        ],
    )(task_ids.astype(jnp.int32), module_logits, x, weight, bias2, la, lb)
    return out
